# baseline (speedup 1.0000x reference)
import dataclasses
import functools

import jax
import jax.numpy as jnp
from jax import lax
from jax.experimental import pallas as pl
from jax.experimental.pallas import tpu as pltpu
from jax.experimental.pallas import tpu_sc as plsc

_N = 50000
_C = 20
_E = 800000
_D = 128
_S = 1024
_T = 0.3
_M_PULL = 0.01
_M_PUSH = 0.2

_NW = 32
_BN = 4096
_NBLK = 13
_NP = _BN * _NBLK
_PPT = _NP // _NW
_PCH = 128
_NCH = _PPT // _PCH

_EPT = 25088
_EP = _NW * _EPT
_ECH = 3136
_NECH = _EPT // _ECH

_SROWS = _S + 16
_U = 288


@functools.lru_cache(maxsize=None)
def _sc_mesh():
    return plsc.VectorSubcoreMesh(core_axis_name="c", subcore_axis_name="s")


def _gwid():
    return lax.axis_index("s") * 2 + lax.axis_index("c")


def _sc_params():
    cp = pltpu.CompilerParams()
    if "needs_layout_passes" in pltpu.CompilerParams.__dataclass_fields__:
        cp = dataclasses.replace(cp, needs_layout_passes=False)
    return cp


def _sc_edgedot(src_pad, dst_pad, x_pad, labels):
    f32 = jnp.float32
    scratch = [
        pltpu.VMEM((_N,), jnp.int32),
        [[pltpu.VMEM((_ECH,), jnp.int32),
          pltpu.VMEM((_ECH,), jnp.int32),
          pltpu.VMEM((_ECH,), f32)] for _ in range(2)],
        pltpu.VMEM((16,), f32),
        [pltpu.SemaphoreType.DMA for _ in range(2)],
        pltpu.SemaphoreType.DMA,
    ]

    @functools.partial(
        pl.kernel, out_type=jax.ShapeDtypeStruct((_NW * 16,), f32),
        mesh=_sc_mesh(), scratch_types=scratch,
        compiler_params=_sc_params())
    def k(src_hbm, dst_hbm, x_hbm, lab_hbm, out_hbm,
          lab_v, bufs, acc_v, sems, lsem):
        gwid = _gwid()
        ldesc = pltpu.async_copy(lab_hbm, lab_v, lsem)
        tile_base = gwid * _EPT

        def stage(ch, b):
            base = tile_base + ch * _ECH
            sv, dv, xv = bufs[b]
            d1 = pltpu.async_copy(src_hbm.at[pl.ds(base, _ECH)], sv, sems[b])
            d2 = pltpu.async_copy(dst_hbm.at[pl.ds(base, _ECH)], dv, sems[b])
            d3 = pltpu.async_copy(x_hbm.at[pl.ds(base, _ECH)], xv, sems[b])
            return (d1, d2, d3)

        descs = [None, None]
        descs[0] = stage(0, 0)
        descs[1] = stage(1, 1)
        ldesc.wait()
        acc_v[...] = jnp.zeros((16,), f32)
        for ch in range(_NECH):
            b = ch % 2
            for d in descs[b]:
                d.wait()
            sv, dv, xv = bufs[b]

            def body(i, acc):
                sl = sv[pl.ds(i * 16, 16)]
                dl = dv[pl.ds(i * 16, 16)]
                ls = plsc.load_gather(lab_v, [sl])
                ld = plsc.load_gather(lab_v, [dl])
                xvv = xv[pl.ds(i * 16, 16)]
                return acc + jnp.where(ls == ld, xvv, jnp.zeros((16,), f32))

            res = lax.fori_loop(0, _ECH // 16, body, jnp.zeros((16,), f32))
            acc_v[...] = acc_v[...] + res
            if ch + 2 < _NECH:
                descs[b] = stage(ch + 2, b)

        pltpu.sync_copy(acc_v, out_hbm.at[pl.ds(gwid * 16, 16)])

    return k(src_pad, dst_pad, x_pad, labels)


_NBUF = 4


def _sc_mugather(mu_a, mu_c, ida_pad, idc_pad):
    f32 = jnp.float32
    out_types = (
        jax.ShapeDtypeStruct((_NP, _D), f32),
        jax.ShapeDtypeStruct((_NP, _D), f32),
    )
    scratch = [
        pltpu.VMEM((_PPT,), jnp.int32),
        pltpu.VMEM((_PPT,), jnp.int32),
        [pltpu.VMEM((_PCH, _D), f32) for _ in range(_NBUF)],
        [pltpu.SemaphoreType.DMA for _ in range(_NBUF)],
        [pltpu.SemaphoreType.DMA for _ in range(_NBUF)],
        pltpu.SemaphoreType.DMA,
    ]

    @functools.partial(pl.kernel, out_type=out_types, mesh=_sc_mesh(),
                       scratch_types=scratch)
    def k(mua_hbm, muc_hbm, ida_hbm, idc_hbm, oa_hbm, oc_hbm,
          idxa_v, idxc_v, bufs, gsems, wsems, isem):
        gwid = _gwid()
        tile_base = gwid * _PPT
        pltpu.async_copy(ida_hbm.at[pl.ds(tile_base, _PPT)], idxa_v, isem)
        pltpu.async_copy(idc_hbm.at[pl.ds(tile_base, _PPT)], idxc_v,
                         isem).wait()

        units = ([(mua_hbm, idxa_v, oa_hbm, ch) for ch in range(_NCH)]
                 + [(muc_hbm, idxc_v, oc_hbm, ch) for ch in range(_NCH)])
        nu = len(units)

        def start_gather(ku):
            tbl, idxv, _, ch = units[ku]
            b = ku % _NBUF
            return pltpu.async_copy(
                tbl.at[idxv.at[pl.ds(ch * _PCH, _PCH)]], bufs[b], gsems[b])

        gd = [None] * nu
        wd = [None] * nu
        for ku in range(min(_NBUF, nu)):
            gd[ku] = start_gather(ku)
        for ku in range(nu):
            _, _, out, ch = units[ku]
            b = ku % _NBUF
            gd[ku].wait()
            base = tile_base + ch * _PCH
            wd[ku] = pltpu.async_copy(bufs[b], out.at[pl.ds(base, _PCH)],
                                      wsems[b])
            if ku + _NBUF < nu:
                wd[ku].wait()
                gd[ku + _NBUF] = start_gather(ku + _NBUF)
        for ku in range(nu - _NBUF, nu):
            wd[ku].wait()

    return k(mu_a, mu_c, ida_pad, idc_pad)


def _tc_segsums(p_pad, ida3, idc3, lab2):
    f32 = jnp.float32
    bf16 = jnp.bfloat16

    def body(p_ref, ida_ref, idc_ref, lab_ref, oa_ref, oc_ref):
        i = pl.program_id(0)
        p = p_ref[...]
        ph = p.astype(bf16)
        plo = (p - ph.astype(f32)).astype(bf16)
        lab = lab_ref[...]
        lane32 = lax.broadcasted_iota(jnp.int32, (_BN, 32), 1)
        laboh = (lane32 == lab).astype(bf16)
        u = jnp.concatenate([ph, plo, laboh], axis=1)
        seg_iota = lax.broadcasted_iota(jnp.int32, (_S, _BN), 0)

        def acc(ids_ref, o_ref):
            idr = ids_ref[0]
            oht = (seg_iota == idr).astype(bf16)
            part = lax.dot_general(oht, u, (((1,), (0,)), ((), ())),
                                   preferred_element_type=f32)

            @pl.when(i == 0)
            def _():
                o_ref[...] = part

            @pl.when(i != 0)
            def _():
                o_ref[...] = o_ref[...] + part

        acc(ida_ref, oa_ref)
        acc(idc_ref, oc_ref)

    return pl.pallas_call(
        body,
        grid=(_NBLK,),
        in_specs=[
            pl.BlockSpec((_BN, _D), lambda i: (i, 0)),
            pl.BlockSpec((1, 1, _BN), lambda i: (i, 0, 0)),
            pl.BlockSpec((1, 1, _BN), lambda i: (i, 0, 0)),
            pl.BlockSpec((_BN, 1), lambda i: (i, 0)),
        ],
        out_specs=(
            pl.BlockSpec((_S, _U), lambda i: (0, 0)),
            pl.BlockSpec((_S, _U), lambda i: (0, 0)),
        ),
        out_shape=(
            jax.ShapeDtypeStruct((_S, _U), f32),
            jax.ShapeDtypeStruct((_S, _U), f32),
        ),
    )(p_pad, ida3, idc3, lab2)


def _tc_ce(seg_logits, labels2d):
    rows = 2000

    def body(lg_ref, lab_ref, out_ref):
        i = pl.program_id(0)
        lg = lg_ref[...]
        lab = lab_ref[...]
        m = jnp.max(lg, axis=1, keepdims=True)
        lse = jnp.log(jnp.sum(jnp.exp(lg - m), axis=1, keepdims=True)) + m
        iot = lax.broadcasted_iota(jnp.int32, (rows, _C), 1)
        picked = jnp.sum(jnp.where(iot == lab, lg, 0.0), axis=1,
                         keepdims=True)
        s = jnp.sum(lse - picked)
        out_ref[...] = jnp.where(i == 0, jnp.full((1, 1), s),
                                 out_ref[...] + s)

    return pl.pallas_call(
        body,
        grid=(_N // rows,),
        in_specs=[
            pl.BlockSpec((rows, _C), lambda i: (i, 0)),
            pl.BlockSpec((rows, 1), lambda i: (i, 0)),
        ],
        out_specs=pl.BlockSpec((1, 1), lambda i: (0, 0)),
        out_shape=jax.ShapeDtypeStruct((1, 1), jnp.float32),
    )(seg_logits, labels2d)


def _tc_dense_edge(x2d):
    def body(x_ref, out_ref):
        x = x_ref[...] / _T
        s = jnp.sum(jnp.maximum(x, 0.0) + jnp.log1p(jnp.exp(-jnp.abs(x))))
        out_ref[...] = jnp.full((1, 1), s)

    return pl.pallas_call(
        body,
        out_shape=jax.ShapeDtypeStruct((1, 1), jnp.float32),
    )(x2d)


def _tc_stats(sta, stc):
    f32 = jnp.float32

    def one(st):
        seg = st[:, 0:_D] + st[:, _D:2 * _D]
        labcnt = st[:, 2 * _D:2 * _D + _C]
        counts = jnp.sum(labcnt, axis=-1, keepdims=True)
        mu = seg / jnp.maximum(counts, 1.0)
        nrm = jnp.sqrt(jnp.sum(mu * mu, axis=-1, keepdims=True))
        mun = mu / (nrm + 1e-8)
        cos = lax.dot_general(mun, mun, (((1,), (1,)), ((), ())),
                              preferred_element_type=f32)
        iot = lax.broadcasted_iota(jnp.int32, (_S, _C), 1)
        mx = jnp.max(labcnt, axis=-1, keepdims=True)
        lblval = jnp.min(jnp.where(labcnt == mx, iot, _C + 7), axis=-1,
                         keepdims=True)
        v = (counts > 0.0).astype(f32)
        w = jnp.where(iot == lblval, v, 0.0)
        h = jnp.maximum(cos - _M_PUSH, 0.0)
        t = lax.dot_general(h, v, (((0,), (0,)), ((), ())),
                            preferred_element_type=f32)
        s_all = jnp.sum(t * v)
        hw = lax.dot_general(h, w, (((1,), (0,)), ((), ())),
                             preferred_element_type=f32)
        s_same = jnp.sum(w * hw)
        push_num = s_all - s_same
        colsum = jnp.sum(w, axis=0, keepdims=True)
        pv_sum = jnp.sum(v) ** 2 - jnp.sum(colsum * colsum)
        return mu, push_num, pv_sum

    def body(sta_ref, stc_ref, mua_ref, muc_ref, scal_ref):
        mu_a, pna, pva = one(sta_ref[...])
        mu_c, pnc, pvc = one(stc_ref[...])
        mua_ref[0:_S, :] = mu_a
        mua_ref[_S:_SROWS, :] = jnp.zeros((_SROWS - _S, _D), f32)
        muc_ref[0:_S, :] = mu_c
        muc_ref[_S:_SROWS, :] = jnp.zeros((_SROWS - _S, _D), f32)
        lane = lax.broadcasted_iota(jnp.int32, (1, 128), 1)
        row = jnp.where(lane == 0, pna,
              jnp.where(lane == 1, pva,
              jnp.where(lane == 2, pnc,
              jnp.where(lane == 3, pvc, 0.0))))
        scal_ref[...] = row

    return pl.pallas_call(
        body,
        out_shape=(
            jax.ShapeDtypeStruct((_SROWS, _D), f32),
            jax.ShapeDtypeStruct((_SROWS, _D), f32),
            jax.ShapeDtypeStruct((1, 128), f32),
        ),
    )(sta, stc)


def _tc_combine(p_pad, ma, mc, ce, dense, dotp, scal):
    rows = 1024
    nblk = _NP // rows

    def body(p_ref, ma_ref, mc_ref, ce_ref, dn_ref, dotp_ref, scal_ref,
             out_ref, acc_ref):
        i = pl.program_id(0)
        p = p_ref[...]
        da = p - ma_ref[...]
        dc = p - mc_ref[...]
        d2a = jnp.sum(da * da, axis=-1, keepdims=True)
        d2c = jnp.sum(dc * dc, axis=-1, keepdims=True)
        sa = jnp.sum(jnp.maximum(d2a - _M_PULL, 0.0))
        sc_ = jnp.sum(jnp.maximum(d2c - _M_PULL, 0.0))

        @pl.when(i == 0)
        def _():
            acc_ref[0] = 0.0
            acc_ref[1] = 0.0

        acc_ref[0] += sa
        acc_ref[1] += sc_

        @pl.when(i == nblk - 1)
        def _():
            dot_tot = jnp.sum(dotp_ref[...])
            ce_s = ce_ref[0, 0]
            dn_s = dn_ref[0, 0]
            pna = scal_ref[0, 0]
            pva = scal_ref[0, 1]
            pnc = scal_ref[0, 2]
            pvc = scal_ref[0, 3]
            nf = jnp.float32(_N)
            ef = jnp.float32(_E)
            loss_ce = ce_s / nf
            loss_cbl = (dn_s - dot_tot / _T) / ef
            disc = (acc_ref[0] / nf + pna / jnp.maximum(pva, 1.0)
                    + acc_ref[1] / nf + pnc / jnp.maximum(pvc, 1.0))
            out_ref[...] = jnp.full((1, 1), loss_ce + loss_cbl + disc)

    return pl.pallas_call(
        body,
        grid=(nblk,),
        in_specs=[
            pl.BlockSpec((rows, _D), lambda i: (i, 0)),
            pl.BlockSpec((rows, _D), lambda i: (i, 0)),
            pl.BlockSpec((rows, _D), lambda i: (i, 0)),
            pl.BlockSpec((1, 1), lambda i: (0, 0),
                         memory_space=pltpu.SMEM),
            pl.BlockSpec((1, 1), lambda i: (0, 0),
                         memory_space=pltpu.SMEM),
            pl.BlockSpec((_NW, 16), lambda i: (0, 0)),
            pl.BlockSpec((1, 128), lambda i: (0, 0),
                         memory_space=pltpu.SMEM),
        ],
        out_specs=pl.BlockSpec((1, 1), lambda i: (0, 0)),
        out_shape=jax.ShapeDtypeStruct((1, 1), jnp.float32),
        scratch_shapes=[pltpu.SMEM((2,), jnp.float32)],
    )(p_pad, ma, mc, ce, dense, dotp, scal)


def kernel(seg_logits, knn_edge_index, knn_edge_logits, p_fea, asso_data,
           cc_ids, labels):
    np_pad = _NP - _N
    p_pad = jnp.pad(p_fea, ((0, np_pad), (0, 0)))
    ida_pad = jnp.pad(asso_data, (0, np_pad), constant_values=_S)
    idc_pad = jnp.pad(cc_ids, (0, np_pad), constant_values=_S)
    lab_pad = jnp.pad(labels, (0, np_pad))
    ep_pad = _EP - _E
    src_pad = jnp.pad(knn_edge_index[0], (0, ep_pad))
    dst_pad = jnp.pad(knn_edge_index[1], (0, ep_pad))
    x_pad = jnp.pad(knn_edge_logits, (0, ep_pad))

    sta, stc = _tc_segsums(p_pad, ida_pad.reshape(_NBLK, 1, _BN),
                           idc_pad.reshape(_NBLK, 1, _BN),
                           lab_pad.reshape(_NP, 1))
    dotp = _sc_edgedot(src_pad, dst_pad, x_pad, labels).reshape(_NW, 16)
    ce = _tc_ce(seg_logits, labels.reshape(_N, 1))
    dense = _tc_dense_edge(knn_edge_logits.reshape(_E // 128, 128))
    mu_a, mu_c, scal = _tc_stats(sta, stc)
    ma, mc = _sc_mugather(mu_a, mu_c, ida_pad, idc_pad)
    total = _tc_combine(p_pad, ma, mc, ce, dense, dotp, scal)
    return jnp.reshape(total, ())


# speedup vs baseline: 1.0849x; 1.0849x over previous
import dataclasses
import functools

import jax
import jax.numpy as jnp
from jax import lax
from jax.experimental import pallas as pl
from jax.experimental.pallas import tpu as pltpu
from jax.experimental.pallas import tpu_sc as plsc

_N = 50000
_C = 20
_E = 800000
_D = 128
_S = 1024
_T = 0.3
_M_PULL = 0.01
_M_PUSH = 0.2

_NW = 32
_BN = 4096
_NBLK = 13
_NP = _BN * _NBLK
_PPT = _NP // _NW
_PCH = 128
_NCH = _PPT // _PCH

_EPT = 25088
_EP = _NW * _EPT
_ECH = 3136
_NECH = _EPT // _ECH

_SROWS = _S + 16
_U = 160


@functools.lru_cache(maxsize=None)
def _sc_mesh():
    return plsc.VectorSubcoreMesh(core_axis_name="c", subcore_axis_name="s")


def _gwid():
    return lax.axis_index("s") * 2 + lax.axis_index("c")


def _sc_params():
    cp = pltpu.CompilerParams()
    if "needs_layout_passes" in pltpu.CompilerParams.__dataclass_fields__:
        cp = dataclasses.replace(cp, needs_layout_passes=False)
    return cp


def _sc_edgedot(src_pad, dst_pad, x_pad, labels):
    f32 = jnp.float32
    scratch = [
        pltpu.VMEM((_N,), jnp.int32),
        [[pltpu.VMEM((_ECH,), jnp.int32),
          pltpu.VMEM((_ECH,), jnp.int32),
          pltpu.VMEM((_ECH,), f32)] for _ in range(2)],
        pltpu.VMEM((16,), f32),
        [pltpu.SemaphoreType.DMA for _ in range(2)],
        pltpu.SemaphoreType.DMA,
    ]

    @functools.partial(
        pl.kernel, out_type=jax.ShapeDtypeStruct((_NW * 16,), f32),
        mesh=_sc_mesh(), scratch_types=scratch,
        compiler_params=_sc_params())
    def k(src_hbm, dst_hbm, x_hbm, lab_hbm, out_hbm,
          lab_v, bufs, acc_v, sems, lsem):
        gwid = _gwid()
        ldesc = pltpu.async_copy(lab_hbm, lab_v, lsem)
        tile_base = gwid * _EPT

        def stage(ch, b):
            base = tile_base + ch * _ECH
            sv, dv, xv = bufs[b]
            d1 = pltpu.async_copy(src_hbm.at[pl.ds(base, _ECH)], sv, sems[b])
            d2 = pltpu.async_copy(dst_hbm.at[pl.ds(base, _ECH)], dv, sems[b])
            d3 = pltpu.async_copy(x_hbm.at[pl.ds(base, _ECH)], xv, sems[b])
            return (d1, d2, d3)

        descs = [None, None]
        descs[0] = stage(0, 0)
        descs[1] = stage(1, 1)
        ldesc.wait()
        acc_v[...] = jnp.zeros((16,), f32)
        for ch in range(_NECH):
            b = ch % 2
            for d in descs[b]:
                d.wait()
            sv, dv, xv = bufs[b]

            def body(i, acc):
                sl = sv[pl.ds(i * 16, 16)]
                dl = dv[pl.ds(i * 16, 16)]
                ls = plsc.load_gather(lab_v, [sl])
                ld = plsc.load_gather(lab_v, [dl])
                xvv = xv[pl.ds(i * 16, 16)]
                return acc + jnp.where(ls == ld, xvv, jnp.zeros((16,), f32))

            res = lax.fori_loop(0, _ECH // 16, body, jnp.zeros((16,), f32))
            acc_v[...] = acc_v[...] + res
            if ch + 2 < _NECH:
                descs[b] = stage(ch + 2, b)

        pltpu.sync_copy(acc_v, out_hbm.at[pl.ds(gwid * 16, 16)])

    return k(src_pad, dst_pad, x_pad, labels)


_NBUF = 6
_LEAD = 3


def _sc_mugather(mu_a, mu_c, ida_pad, idc_pad):
    f32 = jnp.float32
    out_types = (
        jax.ShapeDtypeStruct((_NP, _D), f32),
        jax.ShapeDtypeStruct((_NP, _D), f32),
    )
    scratch = [
        pltpu.VMEM((_PPT,), jnp.int32),
        pltpu.VMEM((_PPT,), jnp.int32),
        [pltpu.VMEM((_PCH, _D), f32) for _ in range(_NBUF)],
        [pltpu.SemaphoreType.DMA for _ in range(_NBUF)],
        [pltpu.SemaphoreType.DMA for _ in range(_NBUF)],
        pltpu.SemaphoreType.DMA,
    ]

    @functools.partial(pl.kernel, out_type=out_types, mesh=_sc_mesh(),
                       scratch_types=scratch)
    def k(mua_hbm, muc_hbm, ida_hbm, idc_hbm, oa_hbm, oc_hbm,
          idxa_v, idxc_v, bufs, gsems, wsems, isem):
        gwid = _gwid()
        tile_base = gwid * _PPT
        pltpu.async_copy(ida_hbm.at[pl.ds(tile_base, _PPT)], idxa_v, isem)
        pltpu.async_copy(idc_hbm.at[pl.ds(tile_base, _PPT)], idxc_v,
                         isem).wait()

        units = ([(mua_hbm, idxa_v, oa_hbm, ch) for ch in range(_NCH)]
                 + [(muc_hbm, idxc_v, oc_hbm, ch) for ch in range(_NCH)])
        nu = len(units)

        def start_gather(ku):
            tbl, idxv, _, ch = units[ku]
            b = ku % _NBUF
            return pltpu.async_copy(
                tbl.at[idxv.at[pl.ds(ch * _PCH, _PCH)]], bufs[b], gsems[b])

        gd = [None] * nu
        wd = [None] * nu
        for ku in range(min(_LEAD, nu)):
            gd[ku] = start_gather(ku)
        for ku in range(nu):
            _, _, out, ch = units[ku]
            b = ku % _NBUF
            gd[ku].wait()
            base = tile_base + ch * _PCH
            wd[ku] = pltpu.async_copy(bufs[b], out.at[pl.ds(base, _PCH)],
                                      wsems[b])
            j = ku + _LEAD
            if j < nu:
                if j - _NBUF >= 0:
                    wd[j - _NBUF].wait()
                gd[j] = start_gather(j)
        for ku in range(max(nu - _NBUF, 0), nu):
            wd[ku].wait()

    return k(mu_a, mu_c, ida_pad, idc_pad)


def _tc_segsums(p_pad, ida3, idc3, lab2):
    f32 = jnp.float32
    bf16 = jnp.bfloat16

    def body(p_ref, ida_ref, idc_ref, lab_ref, oa_ref, oc_ref):
        i = pl.program_id(0)
        ph = p_ref[...].astype(bf16)
        lab = lab_ref[...]
        lane32 = lax.broadcasted_iota(jnp.int32, (_BN, 32), 1)
        laboh = (lane32 == lab).astype(bf16)
        u = jnp.concatenate([ph, laboh], axis=1)
        seg_iota = lax.broadcasted_iota(jnp.int32, (_S, _BN), 0)

        def acc(ids_ref, o_ref):
            idr = ids_ref[0]
            oht = (seg_iota == idr).astype(bf16)
            part = lax.dot_general(oht, u, (((1,), (0,)), ((), ())),
                                   preferred_element_type=f32)

            @pl.when(i == 0)
            def _():
                o_ref[...] = part

            @pl.when(i != 0)
            def _():
                o_ref[...] = o_ref[...] + part

        acc(ida_ref, oa_ref)
        acc(idc_ref, oc_ref)

    return pl.pallas_call(
        body,
        grid=(_NBLK,),
        in_specs=[
            pl.BlockSpec((_BN, _D), lambda i: (i, 0)),
            pl.BlockSpec((1, 1, _BN), lambda i: (i, 0, 0)),
            pl.BlockSpec((1, 1, _BN), lambda i: (i, 0, 0)),
            pl.BlockSpec((_BN, 1), lambda i: (i, 0)),
        ],
        out_specs=(
            pl.BlockSpec((_S, _U), lambda i: (0, 0)),
            pl.BlockSpec((_S, _U), lambda i: (0, 0)),
        ),
        out_shape=(
            jax.ShapeDtypeStruct((_S, _U), f32),
            jax.ShapeDtypeStruct((_S, _U), f32),
        ),
    )(p_pad, ida3, idc3, lab2)


def _tc_ce(logits_t, labels_row):
    def body(lg_ref, lab_ref, out_ref):
        lg = lg_ref[...]
        lab = lab_ref[...]
        m = jnp.max(lg, axis=0, keepdims=True)
        lse = jnp.log(jnp.sum(jnp.exp(lg - m), axis=0, keepdims=True)) + m
        iot = lax.broadcasted_iota(jnp.int32, (_C, _N), 0)
        picked = jnp.sum(jnp.where(iot == lab, lg, 0.0), axis=0,
                         keepdims=True)
        out_ref[...] = jnp.full((1, 1), jnp.sum(lse - picked))

    return pl.pallas_call(
        body,
        out_shape=jax.ShapeDtypeStruct((1, 1), jnp.float32),
    )(logits_t, labels_row)


def _tc_dense_edge(x2d):
    def body(x_ref, out_ref):
        x = x_ref[...] / _T
        s = jnp.sum(jnp.maximum(x, 0.0) + jnp.log1p(jnp.exp(-jnp.abs(x))))
        out_ref[...] = jnp.full((1, 1), s)

    return pl.pallas_call(
        body,
        out_shape=jax.ShapeDtypeStruct((1, 1), jnp.float32),
    )(x2d)


def _tc_stats(sta, stc):
    f32 = jnp.float32

    def one(st):
        seg = st[:, 0:_D]
        labcnt = st[:, _D:_D + _C]
        counts = jnp.sum(labcnt, axis=-1, keepdims=True)
        mu = seg / jnp.maximum(counts, 1.0)
        nrm = jnp.sqrt(jnp.sum(mu * mu, axis=-1, keepdims=True))
        mun = mu / (nrm + 1e-8)
        cos = lax.dot_general(mun, mun, (((1,), (1,)), ((), ())),
                              preferred_element_type=f32)
        iot = lax.broadcasted_iota(jnp.int32, (_S, _C), 1)
        mx = jnp.max(labcnt, axis=-1, keepdims=True)
        lblval = jnp.min(jnp.where(labcnt == mx, iot, _C + 7), axis=-1,
                         keepdims=True)
        v = (counts > 0.0).astype(f32)
        w = jnp.where(iot == lblval, v, 0.0)
        h = jnp.maximum(cos - _M_PUSH, 0.0)
        t = lax.dot_general(h, v, (((0,), (0,)), ((), ())),
                            preferred_element_type=f32)
        s_all = jnp.sum(t * v)
        hw = lax.dot_general(h, w, (((1,), (0,)), ((), ())),
                             preferred_element_type=f32)
        s_same = jnp.sum(w * hw)
        push_num = s_all - s_same
        colsum = jnp.sum(w, axis=0, keepdims=True)
        pv_sum = jnp.sum(v) ** 2 - jnp.sum(colsum * colsum)
        return mu, push_num, pv_sum

    def body(sta_ref, stc_ref, mua_ref, muc_ref, scal_ref):
        mu_a, pna, pva = one(sta_ref[...])
        mu_c, pnc, pvc = one(stc_ref[...])
        mua_ref[0:_S, :] = mu_a
        mua_ref[_S:_SROWS, :] = jnp.zeros((_SROWS - _S, _D), f32)
        muc_ref[0:_S, :] = mu_c
        muc_ref[_S:_SROWS, :] = jnp.zeros((_SROWS - _S, _D), f32)
        lane = lax.broadcasted_iota(jnp.int32, (1, 128), 1)
        row = jnp.where(lane == 0, pna,
              jnp.where(lane == 1, pva,
              jnp.where(lane == 2, pnc,
              jnp.where(lane == 3, pvc, 0.0))))
        scal_ref[...] = row

    return pl.pallas_call(
        body,
        out_shape=(
            jax.ShapeDtypeStruct((_SROWS, _D), f32),
            jax.ShapeDtypeStruct((_SROWS, _D), f32),
            jax.ShapeDtypeStruct((1, 128), f32),
        ),
    )(sta, stc)


def _tc_combine(p_pad, ma, mc, ce, dense, dotp, scal):
    rows = 1024
    nblk = _NP // rows

    def body(p_ref, ma_ref, mc_ref, ce_ref, dn_ref, dotp_ref, scal_ref,
             out_ref, acc_ref):
        i = pl.program_id(0)
        p = p_ref[...]
        da = p - ma_ref[...]
        dc = p - mc_ref[...]
        d2a = jnp.sum(da * da, axis=-1, keepdims=True)
        d2c = jnp.sum(dc * dc, axis=-1, keepdims=True)
        sa = jnp.sum(jnp.maximum(d2a - _M_PULL, 0.0))
        sc_ = jnp.sum(jnp.maximum(d2c - _M_PULL, 0.0))

        @pl.when(i == 0)
        def _():
            acc_ref[0] = 0.0
            acc_ref[1] = 0.0

        acc_ref[0] += sa
        acc_ref[1] += sc_

        @pl.when(i == nblk - 1)
        def _():
            dot_tot = jnp.sum(dotp_ref[...])
            ce_s = ce_ref[0, 0]
            dn_s = dn_ref[0, 0]
            pna = scal_ref[0, 0]
            pva = scal_ref[0, 1]
            pnc = scal_ref[0, 2]
            pvc = scal_ref[0, 3]
            nf = jnp.float32(_N)
            ef = jnp.float32(_E)
            loss_ce = ce_s / nf
            loss_cbl = (dn_s - dot_tot / _T) / ef
            disc = (acc_ref[0] / nf + pna / jnp.maximum(pva, 1.0)
                    + acc_ref[1] / nf + pnc / jnp.maximum(pvc, 1.0))
            out_ref[...] = jnp.full((1, 1), loss_ce + loss_cbl + disc)

    return pl.pallas_call(
        body,
        grid=(nblk,),
        in_specs=[
            pl.BlockSpec((rows, _D), lambda i: (i, 0)),
            pl.BlockSpec((rows, _D), lambda i: (i, 0)),
            pl.BlockSpec((rows, _D), lambda i: (i, 0)),
            pl.BlockSpec((1, 1), lambda i: (0, 0),
                         memory_space=pltpu.SMEM),
            pl.BlockSpec((1, 1), lambda i: (0, 0),
                         memory_space=pltpu.SMEM),
            pl.BlockSpec((_NW, 16), lambda i: (0, 0)),
            pl.BlockSpec((1, 128), lambda i: (0, 0),
                         memory_space=pltpu.SMEM),
        ],
        out_specs=pl.BlockSpec((1, 1), lambda i: (0, 0)),
        out_shape=jax.ShapeDtypeStruct((1, 1), jnp.float32),
        scratch_shapes=[pltpu.SMEM((2,), jnp.float32)],
    )(p_pad, ma, mc, ce, dense, dotp, scal)


def kernel(seg_logits, knn_edge_index, knn_edge_logits, p_fea, asso_data,
           cc_ids, labels):
    np_pad = _NP - _N
    p_pad = jnp.pad(p_fea, ((0, np_pad), (0, 0)))
    ida_pad = jnp.pad(asso_data, (0, np_pad), constant_values=_S)
    idc_pad = jnp.pad(cc_ids, (0, np_pad), constant_values=_S)
    lab_pad = jnp.pad(labels, (0, np_pad))
    ep_pad = _EP - _E
    src_pad = jnp.pad(knn_edge_index[0], (0, ep_pad))
    dst_pad = jnp.pad(knn_edge_index[1], (0, ep_pad))
    x_pad = jnp.pad(knn_edge_logits, (0, ep_pad))

    sta, stc = _tc_segsums(p_pad, ida_pad.reshape(_NBLK, 1, _BN),
                           idc_pad.reshape(_NBLK, 1, _BN),
                           lab_pad.reshape(_NP, 1))
    dotp = _sc_edgedot(src_pad, dst_pad, x_pad, labels).reshape(_NW, 16)
    ce = _tc_ce(jnp.transpose(seg_logits), labels.reshape(1, _N))
    dense = _tc_dense_edge(knn_edge_logits.reshape(_E // 128, 128))
    mu_a, mu_c, scal = _tc_stats(sta, stc)
    ma, mc = _sc_mugather(mu_a, mu_c, ida_pad, idc_pad)
    total = _tc_combine(p_pad, ma, mc, ce, dense, dotp, scal)
    return jnp.reshape(total, ())


# speedup vs baseline: 1.4597x; 1.3454x over previous
import dataclasses
import functools

import jax
import jax.numpy as jnp
from jax import lax
from jax.experimental import pallas as pl
from jax.experimental.pallas import tpu as pltpu
from jax.experimental.pallas import tpu_sc as plsc

_N = 50000
_C = 20
_E = 800000
_D = 128
_S = 1024
_T = 0.3
_M_PULL = 0.01
_M_PUSH = 0.2

_NW = 32
_BN = 4096
_NBLK = 13
_NP = _BN * _NBLK
_PPT = _NP // _NW
_PCH = 128
_NCH = _PPT // _PCH

_EPT = 25088
_EP = _NW * _EPT
_ECH = 3136
_NECH = _EPT // _ECH

_SROWS = _S + 16
_U = 160


@functools.lru_cache(maxsize=None)
def _sc_mesh():
    return plsc.VectorSubcoreMesh(core_axis_name="c", subcore_axis_name="s")


def _gwid():
    return lax.axis_index("s") * 2 + lax.axis_index("c")


def _sc_params():
    cp = pltpu.CompilerParams()
    if "needs_layout_passes" in pltpu.CompilerParams.__dataclass_fields__:
        cp = dataclasses.replace(cp, needs_layout_passes=False)
    return cp


def _sc_edgedot(src_pad, dst_pad, x_pad, labels):
    f32 = jnp.float32
    scratch = [
        pltpu.VMEM((_N,), jnp.int32),
        [[pltpu.VMEM((_ECH,), jnp.int32),
          pltpu.VMEM((_ECH,), jnp.int32),
          pltpu.VMEM((_ECH,), f32)] for _ in range(2)],
        pltpu.VMEM((16,), f32),
        [pltpu.SemaphoreType.DMA for _ in range(2)],
        pltpu.SemaphoreType.DMA,
    ]

    @functools.partial(
        pl.kernel, out_type=jax.ShapeDtypeStruct((_NW * 16,), f32),
        mesh=_sc_mesh(), scratch_types=scratch,
        compiler_params=_sc_params())
    def k(src_hbm, dst_hbm, x_hbm, lab_hbm, out_hbm,
          lab_v, bufs, acc_v, sems, lsem):
        gwid = _gwid()
        ldesc = pltpu.async_copy(lab_hbm, lab_v, lsem)
        tile_base = gwid * _EPT

        def stage(ch, b):
            base = tile_base + ch * _ECH
            sv, dv, xv = bufs[b]
            d1 = pltpu.async_copy(src_hbm.at[pl.ds(base, _ECH)], sv, sems[b])
            d2 = pltpu.async_copy(dst_hbm.at[pl.ds(base, _ECH)], dv, sems[b])
            d3 = pltpu.async_copy(x_hbm.at[pl.ds(base, _ECH)], xv, sems[b])
            return (d1, d2, d3)

        descs = [None, None]
        descs[0] = stage(0, 0)
        descs[1] = stage(1, 1)
        ldesc.wait()
        acc_v[...] = jnp.zeros((16,), f32)
        for ch in range(_NECH):
            b = ch % 2
            for d in descs[b]:
                d.wait()
            sv, dv, xv = bufs[b]

            def body(i, acc):
                sl = sv[pl.ds(i * 16, 16)]
                dl = dv[pl.ds(i * 16, 16)]
                ls = plsc.load_gather(lab_v, [sl])
                ld = plsc.load_gather(lab_v, [dl])
                xvv = xv[pl.ds(i * 16, 16)]
                return acc + jnp.where(ls == ld, xvv, jnp.zeros((16,), f32))

            res = lax.fori_loop(0, _ECH // 16, body, jnp.zeros((16,), f32))
            acc_v[...] = acc_v[...] + res
            if ch + 2 < _NECH:
                descs[b] = stage(ch + 2, b)

        pltpu.sync_copy(acc_v, out_hbm.at[pl.ds(gwid * 16, 16)])

    return k(src_pad, dst_pad, x_pad, labels)


def _sc_pull(mu_a, mu_c, ida_pad, idc_pad, p_pad):
    f32 = jnp.float32
    i32 = jnp.int32
    out_types = (
        jax.ShapeDtypeStruct((_NP,), f32),
        jax.ShapeDtypeStruct((_NP,), f32),
    )
    scratch = [
        pltpu.VMEM((_PPT,), i32),
        pltpu.VMEM((_PPT,), i32),
        [pltpu.VMEM((_PCH, _D), f32) for _ in range(2)],
        [pltpu.VMEM((_PCH, _D), f32) for _ in range(3)],
        [pltpu.VMEM((_PCH, _D), f32) for _ in range(2)],
        pltpu.VMEM((256,), f32),
        pltpu.VMEM((_PPT,), f32),
        pltpu.VMEM((_PPT,), f32),
        [pltpu.SemaphoreType.DMA for _ in range(2)],
        [pltpu.SemaphoreType.DMA for _ in range(3)],
        [pltpu.SemaphoreType.DMA for _ in range(2)],
        pltpu.SemaphoreType.DMA,
    ]

    @functools.partial(pl.kernel, out_type=out_types, mesh=_sc_mesh(),
                       scratch_types=scratch,
                       compiler_params=_sc_params())
    def k(mua_hbm, muc_hbm, ida_hbm, idc_hbm, p_hbm, oa_hbm, oc_hbm,
          idxa_v, idxc_v, pbufs, gabufs, gcbufs, t_v, d2a_v, d2c_v,
          psems, gasems, gcsems, isem):
        gwid = _gwid()
        tile_base = gwid * _PPT
        da = pltpu.async_copy(ida_hbm.at[pl.ds(tile_base, _PPT)], idxa_v,
                              isem)
        dc = pltpu.async_copy(idc_hbm.at[pl.ds(tile_base, _PPT)], idxc_v,
                              isem)
        da.wait()
        dc.wait()
        iota16 = lax.iota(i32, 16) * 16

        def issue_ga(ch):
            return pltpu.async_copy(
                mua_hbm.at[idxa_v.at[pl.ds(ch * _PCH, _PCH)]],
                gabufs[ch % 3], gasems[ch % 3])

        def issue_pgc(ch):
            base = tile_base + ch * _PCH
            pd = pltpu.async_copy(p_hbm.at[pl.ds(base, _PCH)],
                                  pbufs[ch % 2], psems[ch % 2])
            gc = pltpu.async_copy(
                muc_hbm.at[idxc_v.at[pl.ds(ch * _PCH, _PCH)]],
                gcbufs[ch % 2], gcsems[ch % 2])
            return pd, gc

        def compute(pbuf, mbuf, d2_v, ch):
            @pl.loop(0, _PCH // 16)
            def _(g):
                @pl.loop(0, 16)
                def _(li):
                    i = g * 16 + li
                    acc = jnp.zeros((16,), f32)
                    for kk in range(8):
                        dp = (pbuf[i, pl.ds(kk * 16, 16)]
                              - mbuf[i, pl.ds(kk * 16, 16)])
                        acc = acc + dp * dp
                    t_v[pl.ds(li * 16, 16)] = acc

                def red(l, d2):
                    return d2 + plsc.load_gather(t_v, [iota16 + l])

                d2vec = lax.fori_loop(0, 16, red, jnp.zeros((16,), f32))
                d2_v[pl.ds(ch * _PCH + g * 16, 16)] = d2vec

        gad = [None] * _NCH
        pgcd = [None] * _NCH
        gad[0] = issue_ga(0)
        gad[1] = issue_ga(1)
        gad[2] = issue_ga(2)
        pgcd[0] = issue_pgc(0)
        pgcd[1] = issue_pgc(1)
        for ch in range(_NCH):
            pd, gc = pgcd[ch]
            pd.wait()
            gad[ch].wait()
            compute(pbufs[ch % 2], gabufs[ch % 3], d2a_v, ch)
            if ch + 3 < _NCH:
                gad[ch + 3] = issue_ga(ch + 3)
            gc.wait()
            compute(pbufs[ch % 2], gcbufs[ch % 2], d2c_v, ch)
            if ch + 2 < _NCH:
                pgcd[ch + 2] = issue_pgc(ch + 2)

        pltpu.sync_copy(d2a_v, oa_hbm.at[pl.ds(tile_base, _PPT)])
        pltpu.sync_copy(d2c_v, oc_hbm.at[pl.ds(tile_base, _PPT)])

    return k(mu_a, mu_c, ida_pad, idc_pad, p_pad)


def _tc_segsums(p_pad, ida3, idc3, lab2):
    f32 = jnp.float32
    bf16 = jnp.bfloat16

    def body(p_ref, ida_ref, idc_ref, lab_ref, oa_ref, oc_ref):
        i = pl.program_id(0)
        ph = p_ref[...].astype(bf16)
        lab = lab_ref[...]
        lane32 = lax.broadcasted_iota(jnp.int32, (_BN, 32), 1)
        laboh = (lane32 == lab).astype(bf16)
        u = jnp.concatenate([ph, laboh], axis=1)
        seg_iota = lax.broadcasted_iota(jnp.int32, (_S, _BN), 0)

        def acc(ids_ref, o_ref):
            idr = ids_ref[0]
            oht = (seg_iota == idr).astype(bf16)
            part = lax.dot_general(oht, u, (((1,), (0,)), ((), ())),
                                   preferred_element_type=f32)

            @pl.when(i == 0)
            def _():
                o_ref[...] = part

            @pl.when(i != 0)
            def _():
                o_ref[...] = o_ref[...] + part

        acc(ida_ref, oa_ref)
        acc(idc_ref, oc_ref)

    return pl.pallas_call(
        body,
        grid=(_NBLK,),
        in_specs=[
            pl.BlockSpec((_BN, _D), lambda i: (i, 0)),
            pl.BlockSpec((1, 1, _BN), lambda i: (i, 0, 0)),
            pl.BlockSpec((1, 1, _BN), lambda i: (i, 0, 0)),
            pl.BlockSpec((_BN, 1), lambda i: (i, 0)),
        ],
        out_specs=(
            pl.BlockSpec((_S, _U), lambda i: (0, 0)),
            pl.BlockSpec((_S, _U), lambda i: (0, 0)),
        ),
        out_shape=(
            jax.ShapeDtypeStruct((_S, _U), f32),
            jax.ShapeDtypeStruct((_S, _U), f32),
        ),
    )(p_pad, ida3, idc3, lab2)


def _tc_ce(logits_t, labels_row):
    def body(lg_ref, lab_ref, out_ref):
        lg = lg_ref[...]
        lab = lab_ref[...]
        m = jnp.max(lg, axis=0, keepdims=True)
        lse = jnp.log(jnp.sum(jnp.exp(lg - m), axis=0, keepdims=True)) + m
        iot = lax.broadcasted_iota(jnp.int32, (_C, _N), 0)
        picked = jnp.sum(jnp.where(iot == lab, lg, 0.0), axis=0,
                         keepdims=True)
        out_ref[...] = jnp.full((1, 1), jnp.sum(lse - picked))

    return pl.pallas_call(
        body,
        out_shape=jax.ShapeDtypeStruct((1, 1), jnp.float32),
    )(logits_t, labels_row)


def _tc_dense_edge(x2d):
    def body(x_ref, out_ref):
        x = x_ref[...] / _T
        s = jnp.sum(jnp.maximum(x, 0.0) + jnp.log1p(jnp.exp(-jnp.abs(x))))
        out_ref[...] = jnp.full((1, 1), s)

    return pl.pallas_call(
        body,
        out_shape=jax.ShapeDtypeStruct((1, 1), jnp.float32),
    )(x2d)


def _tc_stats(sta, stc):
    f32 = jnp.float32

    def one(st):
        seg = st[:, 0:_D]
        labcnt = st[:, _D:_D + _C]
        counts = jnp.sum(labcnt, axis=-1, keepdims=True)
        mu = seg / jnp.maximum(counts, 1.0)
        nrm = jnp.sqrt(jnp.sum(mu * mu, axis=-1, keepdims=True))
        mun = mu / (nrm + 1e-8)
        cos = lax.dot_general(mun, mun, (((1,), (1,)), ((), ())),
                              preferred_element_type=f32)
        iot = lax.broadcasted_iota(jnp.int32, (_S, _C), 1)
        mx = jnp.max(labcnt, axis=-1, keepdims=True)
        lblval = jnp.min(jnp.where(labcnt == mx, iot, _C + 7), axis=-1,
                         keepdims=True)
        v = (counts > 0.0).astype(f32)
        w = jnp.where(iot == lblval, v, 0.0)
        h = jnp.maximum(cos - _M_PUSH, 0.0)
        t = lax.dot_general(h, v, (((0,), (0,)), ((), ())),
                            preferred_element_type=f32)
        s_all = jnp.sum(t * v)
        hw = lax.dot_general(h, w, (((1,), (0,)), ((), ())),
                             preferred_element_type=f32)
        s_same = jnp.sum(w * hw)
        push_num = s_all - s_same
        colsum = jnp.sum(w, axis=0, keepdims=True)
        pv_sum = jnp.sum(v) ** 2 - jnp.sum(colsum * colsum)
        return mu, push_num, pv_sum

    def body(sta_ref, stc_ref, mua_ref, muc_ref, scal_ref):
        mu_a, pna, pva = one(sta_ref[...])
        mu_c, pnc, pvc = one(stc_ref[...])
        mua_ref[0:_S, :] = mu_a
        mua_ref[_S:_SROWS, :] = jnp.zeros((_SROWS - _S, _D), f32)
        muc_ref[0:_S, :] = mu_c
        muc_ref[_S:_SROWS, :] = jnp.zeros((_SROWS - _S, _D), f32)
        lane = lax.broadcasted_iota(jnp.int32, (1, 128), 1)
        row = jnp.where(lane == 0, pna,
              jnp.where(lane == 1, pva,
              jnp.where(lane == 2, pnc,
              jnp.where(lane == 3, pvc, 0.0))))
        scal_ref[...] = row

    return pl.pallas_call(
        body,
        out_shape=(
            jax.ShapeDtypeStruct((_SROWS, _D), f32),
            jax.ShapeDtypeStruct((_SROWS, _D), f32),
            jax.ShapeDtypeStruct((1, 128), f32),
        ),
    )(sta, stc)


def _tc_combine(d2a2, d2c2, ce, dense, dotp, scal):
    def body(d2a_ref, d2c_ref, ce_ref, dn_ref, dotp_ref, scal_ref, out_ref):
        sa = jnp.sum(jnp.maximum(d2a_ref[...] - _M_PULL, 0.0))
        sc_ = jnp.sum(jnp.maximum(d2c_ref[...] - _M_PULL, 0.0))
        dot_tot = jnp.sum(dotp_ref[...])
        ce_s = ce_ref[0, 0]
        dn_s = dn_ref[0, 0]
        pna = scal_ref[0, 0]
        pva = scal_ref[0, 1]
        pnc = scal_ref[0, 2]
        pvc = scal_ref[0, 3]
        nf = jnp.float32(_N)
        ef = jnp.float32(_E)
        loss_ce = ce_s / nf
        loss_cbl = (dn_s - dot_tot / _T) / ef
        disc = (sa / nf + pna / jnp.maximum(pva, 1.0)
                + sc_ / nf + pnc / jnp.maximum(pvc, 1.0))
        out_ref[...] = jnp.full((1, 1), loss_ce + loss_cbl + disc)

    return pl.pallas_call(
        body,
        in_specs=[
            pl.BlockSpec(),
            pl.BlockSpec(),
            pl.BlockSpec(memory_space=pltpu.SMEM),
            pl.BlockSpec(memory_space=pltpu.SMEM),
            pl.BlockSpec(),
            pl.BlockSpec(memory_space=pltpu.SMEM),
        ],
        out_shape=jax.ShapeDtypeStruct((1, 1), jnp.float32),
    )(d2a2, d2c2, ce, dense, dotp, scal)


def kernel(seg_logits, knn_edge_index, knn_edge_logits, p_fea, asso_data,
           cc_ids, labels):
    np_pad = _NP - _N
    p_pad = jnp.pad(p_fea, ((0, np_pad), (0, 0)))
    ida_pad = jnp.pad(asso_data, (0, np_pad), constant_values=_S)
    idc_pad = jnp.pad(cc_ids, (0, np_pad), constant_values=_S)
    lab_pad = jnp.pad(labels, (0, np_pad))
    ep_pad = _EP - _E
    src_pad = jnp.pad(knn_edge_index[0], (0, ep_pad))
    dst_pad = jnp.pad(knn_edge_index[1], (0, ep_pad))
    x_pad = jnp.pad(knn_edge_logits, (0, ep_pad))

    sta, stc = _tc_segsums(p_pad, ida_pad.reshape(_NBLK, 1, _BN),
                           idc_pad.reshape(_NBLK, 1, _BN),
                           lab_pad.reshape(_NP, 1))
    dotp = _sc_edgedot(src_pad, dst_pad, x_pad, labels).reshape(_NW, 16)
    ce = _tc_ce(jnp.transpose(seg_logits), labels.reshape(1, _N))
    dense = _tc_dense_edge(knn_edge_logits.reshape(_E // 128, 128))
    mu_a, mu_c, scal = _tc_stats(sta, stc)
    d2a, d2c = _sc_pull(mu_a, mu_c, ida_pad, idc_pad, p_pad)
    total = _tc_combine(d2a.reshape(_NP // 128, 128),
                        d2c.reshape(_NP // 128, 128), ce, dense, dotp, scal)
    return jnp.reshape(total, ())
